# baseline (speedup 1.0000x reference)
import functools

import jax
import jax.numpy as jnp
from jax import lax
from jax.experimental import pallas as pl
from jax.experimental.pallas import tpu as pltpu
from jax.experimental.pallas import tpu_sc as plsc

NC = 2
NS = 16
NW = NC * NS
LANES = 16

VOCAB = 100000
MAXLEN = 200
EMBED = 64
BATCH = 4096

BPW = BATCH // NW
assert BATCH % NW == 0 and BPW % LANES == 0 and MAXLEN % 2 == 0
KCH = EMBED // LANES


def _emb_body(xt_hbm, tok_hbm, pos_hbm, out_hbm,
              idx_v, rows0, rows1, ob0, ob1, pos_v,
              gsem0, gsem1, osem0, osem1):
    rows = (rows0, rows1)
    ob = (ob0, ob1)
    gsem = (gsem0, gsem1)
    osem = (osem0, osem1)
    wid = lax.axis_index("s") * NC + lax.axis_index("c")
    b0 = wid * BPW
    pltpu.sync_copy(pos_hbm, pos_v)
    pltpu.sync_copy(xt_hbm.at[:, pl.ds(b0, BPW)], idx_v)
    iota = lax.iota(jnp.int32, LANES)

    def start_gather(l, b):
        pltpu.async_copy(tok_hbm.at[idx_v.at[l]], rows[b], gsem[b])

    def wait_gather(b):
        pltpu.make_async_copy(tok_hbm.at[idx_v.at[0]], rows[b], gsem[b]).wait()

    def transpose_add(l, b):
        rv, ov = rows[b], ob[b]
        pvec = [pos_v[l, pl.ds(k * LANES, LANES)] for k in range(KCH)]

        def _tok(t, carry):
            col = iota * 0 + t
            for k in range(KCH):
                v = rv[t, pl.ds(k * LANES, LANES)] + pvec[k]
                plsc.store_scatter(ov, [k * LANES + iota, col], v)
            return carry

        lax.fori_loop(0, BPW, _tok, None)

    def start_write(l, b):
        pltpu.async_copy(ob[b], out_hbm.at[l, :, pl.ds(b0, BPW)], osem[b])

    def wait_write(b):
        pltpu.make_async_copy(ob[b], out_hbm.at[0, :, pl.ds(b0, BPW)], osem[b]).wait()

    start_gather(0, 0)
    wait_gather(0)
    start_gather(1, 1)
    transpose_add(0, 0)
    start_write(0, 0)

    @pl.loop(0, (MAXLEN - 2) // 2)
    def _steady(t):
        for b in (1, 0):
            l = 1 + 2 * t + (0 if b == 1 else 1)
            wait_gather(b)
            obuf = 1 - b
            wait_write(obuf)
            start_gather(l + 1, obuf)
            transpose_add(l, b)
            start_write(l, b)

    wait_gather(1)
    transpose_add(MAXLEN - 1, 1)
    start_write(MAXLEN - 1, 1)
    wait_write(0)
    wait_write(1)


_emb = functools.partial(
    pl.kernel,
    out_type=jax.ShapeDtypeStruct((MAXLEN, EMBED, BATCH), jnp.float32),
    mesh=plsc.VectorSubcoreMesh(core_axis_name="c", subcore_axis_name="s"),
    scratch_types=[
        pltpu.VMEM((MAXLEN, BPW), jnp.int32),
        pltpu.VMEM((BPW, EMBED), jnp.float32),
        pltpu.VMEM((BPW, EMBED), jnp.float32),
        pltpu.VMEM((EMBED, BPW), jnp.float32),
        pltpu.VMEM((EMBED, BPW), jnp.float32),
        pltpu.VMEM((MAXLEN, EMBED), jnp.float32),
        pltpu.SemaphoreType.DMA,
        pltpu.SemaphoreType.DMA,
        pltpu.SemaphoreType.DMA,
        pltpu.SemaphoreType.DMA,
    ],
    compiler_params=pltpu.CompilerParams(
        use_tc_tiling_on_sc=False, needs_layout_passes=False),
)(_emb_body)


def kernel(x, token_table, pos_table):
    xt = x.astype(jnp.int32).T
    out_t = _emb(xt, token_table, pos_table)
    return out_t.transpose(2, 0, 1)


# speedup vs baseline: 1.0053x; 1.0053x over previous
import functools

import jax
import jax.numpy as jnp
from jax import lax
from jax.experimental import pallas as pl
from jax.experimental.pallas import tpu as pltpu
from jax.experimental.pallas import tpu_sc as plsc

NC = 2
NS = 16
NW = NC * NS
LANES = 16

VOCAB = 100000
MAXLEN = 200
EMBED = 64
EPAD = 128
BATCH = 4096

BPW = BATCH // NW
TB = BPW // LANES
assert BATCH % NW == 0 and BPW == 128 and MAXLEN % 2 == 0


def _emb_body(xt_hbm, tok_hbm, pos_hbm, out_hbm,
              idx_v, pos_v, rows0, rows1, ob0, ob1,
              gsem0, gsem1, osem0, osem1):
    rows = (rows0, rows1)
    ob = (ob0, ob1)
    gsem = (gsem0, gsem1)
    osem = (osem0, osem1)
    wid = lax.axis_index("s") * NC + lax.axis_index("c")
    b0 = wid * BPW
    pltpu.sync_copy(pos_hbm, pos_v)
    pltpu.sync_copy(xt_hbm.at[:, pl.ds(b0, BPW)], idx_v)
    iota = lax.iota(jnp.int32, LANES)
    zero = iota * 0

    def start_gather(l, b):
        pltpu.async_copy(tok_hbm.at[idx_v.at[l]], rows[b], gsem[b])

    def wait_gather(b):
        pltpu.make_async_copy(tok_hbm.at[idx_v.at[0]], rows[b], gsem[b]).wait()

    def transpose_add(l, b):
        rv, ov = rows[b], ob[b]
        sl = zero + l

        def _col(e, carry):
            se = zero + e
            pe = plsc.load_gather(pos_v, [sl, se])
            for t in range(TB):
                v = plsc.load_gather(rv, [t * LANES + iota, se])
                ov[e, pl.ds(t * LANES, LANES)] = v + pe
            return carry

        lax.fori_loop(0, EMBED, _col, None)

    def start_write(l, b):
        pltpu.async_copy(ob[b], out_hbm.at[l, :, pl.ds(b0, BPW)], osem[b])

    def wait_write(b):
        pltpu.make_async_copy(ob[b], out_hbm.at[0, :, pl.ds(b0, BPW)], osem[b]).wait()

    start_gather(0, 0)
    wait_gather(0)
    start_gather(1, 1)
    transpose_add(0, 0)
    start_write(0, 0)

    @pl.loop(0, (MAXLEN - 2) // 2)
    def _steady(t):
        for b in (1, 0):
            l = 1 + 2 * t + (0 if b == 1 else 1)
            wait_gather(b)
            obuf = 1 - b
            wait_write(obuf)
            start_gather(l + 1, obuf)
            transpose_add(l, b)
            start_write(l, b)

    wait_gather(1)
    transpose_add(MAXLEN - 1, 1)
    start_write(MAXLEN - 1, 1)
    wait_write(0)
    wait_write(1)


_emb = functools.partial(
    pl.kernel,
    out_type=jax.ShapeDtypeStruct((MAXLEN, EMBED, BATCH), jnp.float32),
    mesh=plsc.VectorSubcoreMesh(core_axis_name="c", subcore_axis_name="s"),
    scratch_types=[
        pltpu.VMEM((MAXLEN, BPW), jnp.int32),
        pltpu.VMEM((MAXLEN, EPAD), jnp.float32),
        pltpu.VMEM((BPW, EPAD), jnp.float32),
        pltpu.VMEM((BPW, EPAD), jnp.float32),
        pltpu.VMEM((EMBED, BPW), jnp.float32),
        pltpu.VMEM((EMBED, BPW), jnp.float32),
        pltpu.SemaphoreType.DMA,
        pltpu.SemaphoreType.DMA,
        pltpu.SemaphoreType.DMA,
        pltpu.SemaphoreType.DMA,
    ],
    compiler_params=pltpu.CompilerParams(
        use_tc_tiling_on_sc=True, needs_layout_passes=False),
)(_emb_body)


def kernel(x, token_table, pos_table):
    xt = x.astype(jnp.int32).T
    tok_pad = jnp.pad(token_table, ((0, 0), (0, EPAD - EMBED)))
    pos_pad = jnp.pad(pos_table, ((0, 0), (0, EPAD - EMBED)))
    out_t = _emb(xt, tok_pad, pos_pad)
    return out_t.transpose(2, 0, 1)
